# Initial kernel scaffold; baseline (speedup 1.0000x reference)
import jax
import jax.numpy as jnp
from jax.experimental import pallas as pl


def kernel(state_node, state_edge, edge_index, phi_W, phi_b, gam_W, gam_b):
    raise NotImplementedError("write your pallas kernel here")


# speedup vs baseline: 3.2865x; 3.2865x over previous
import functools

import jax
import jax.numpy as jnp
from jax import lax
from jax.experimental import pallas as pl
from jax.experimental.pallas import tpu as pltpu
from jax.experimental.pallas import tpu_sc as plsc

N = 10000
E = 320000
D = 128
N_PAD = 10240
E_PAD = 327680
N_TILES = 16
N_CORES = 2
ROWS_PER_TILE = N_PAD // N_TILES
CHUNK = 128
CHUNKS_PER_TILE = E_PAD // (N_TILES * N_CORES * CHUNK)
f32 = jnp.float32


def _make_sc_segsum():
    mesh = plsc.VectorSubcoreMesh(core_axis_name="c", subcore_axis_name="s")

    @functools.partial(
        pl.kernel, mesh=mesh,
        out_type=jax.ShapeDtypeStruct((N_CORES, N_PAD, D), f32),
        scratch_types=[
            pltpu.VMEM((CHUNKS_PER_TILE, CHUNK), jnp.int32),
            pltpu.VMEM((CHUNKS_PER_TILE, CHUNK), jnp.int32),
            pltpu.VMEM((CHUNK, D), f32),
            pltpu.VMEM_SHARED((N_PAD, D), f32),
            pltpu.SemaphoreType.DMA,
        ])
    def sc_kernel(tab_hbm, idx3, row3, zs, s_out, idxbuf, rowbuf, rows_v,
                  sacc, sem):
        cid = lax.axis_index("c")
        sid = lax.axis_index("s")
        wid = cid * N_TILES + sid
        rbase = sid * ROWS_PER_TILE
        pltpu.sync_copy(zs, sacc.at[pl.ds(rbase, ROWS_PER_TILE)])
        pltpu.sync_copy(idx3.at[wid], idxbuf)
        pltpu.sync_copy(row3.at[wid], rowbuf)
        plsc.subcore_barrier()

        def chunk_body(j, carry):
            pltpu.async_copy(tab_hbm.at[idxbuf.at[j]], rows_v, sem).wait()
            pltpu.sync_copy(rows_v, sacc.at[rowbuf.at[j]], add=True)
            return carry

        lax.fori_loop(0, CHUNKS_PER_TILE, chunk_body, 0)
        plsc.subcore_barrier()
        pltpu.sync_copy(sacc.at[pl.ds(rbase, ROWS_PER_TILE)],
                        s_out.at[cid, pl.ds(rbase, ROWS_PER_TILE)])

    return sc_kernel


def _tc_layer(h, s0, s1, e0, e1, ab, bw, cx, g0, g1, gb8, residual: bool):
    BR = 1024
    grid = (N_PAD // BR,)

    def body(h_ref, s0_ref, s1_ref, e0_ref, e1_ref, ab_ref, bw_ref, cx_ref,
             g0_ref, g1_ref, gb_ref, out_ref):
        eb = e0_ref[...] + e1_ref[...]
        cnt = eb[:, 16:17]
        inv = 1.0 / jnp.maximum(cnt, 1.0)
        msk = (cnt > 0.0).astype(f32)
        sb = (s0_ref[...] + s1_ref[...]) * inv
        me = eb * inv
        hv = h_ref[...]
        aggr = (jnp.dot(hv, ab_ref[...], preferred_element_type=f32)
                + jnp.dot(sb, bw_ref[...], preferred_element_type=f32)
                + jnp.dot(me, cx_ref[...], preferred_element_type=f32)) * msk
        out = (jnp.dot(hv, g0_ref[...], preferred_element_type=f32)
               + jnp.dot(aggr, g1_ref[...], preferred_element_type=f32)
               + gb_ref[0:1, :])
        out_ref[...] = out + hv if residual else out

    rowspec = pl.BlockSpec((BR, D), lambda i: (i, 0))
    wspec = pl.BlockSpec((D, D), lambda i: (0, 0))
    return pl.pallas_call(
        body,
        grid=grid,
        in_specs=[rowspec, rowspec, rowspec, rowspec, rowspec,
                  wspec, wspec, wspec,
                  wspec, wspec, pl.BlockSpec((8, D), lambda i: (0, 0))],
        out_specs=rowspec,
        out_shape=jax.ShapeDtypeStruct((N_PAD, D), f32),
    )(h, s0, s1, e0, e1, ab, bw, cx, g0, g1, gb8)


def kernel(state_node, state_edge, edge_index, phi_W, phi_b, gam_W, gam_b):
    L = phi_W.shape[0]
    h = jnp.pad(state_node, ((0, N_PAD - N), (0, 0)))
    pad = E_PAD - E
    row3 = jnp.pad(edge_index[0], (0, pad), constant_values=N).reshape(
        N_TILES * N_CORES, CHUNKS_PER_TILE, CHUNK)
    col3 = jnp.pad(edge_index[1], (0, pad), constant_values=N).reshape(
        N_TILES * N_CORES, CHUNKS_PER_TILE, CHUNK)
    eidx3 = jnp.arange(E_PAD, dtype=jnp.int32).reshape(
        N_TILES * N_CORES, CHUNKS_PER_TILE, CHUNK)
    e2 = jnp.zeros((E_PAD, D), f32)
    e2 = e2.at[:E, :16].set(state_edge).at[:E, 16].set(1.0)
    zs = jnp.zeros((ROWS_PER_TILE, D), f32)

    ab = phi_W[:, :D, :] - phi_W[:, D:2 * D, :]
    bw = phi_W[:, D:2 * D, :]
    cx = jnp.zeros((L, D, D), f32)
    cx = cx.at[:, :16, :].set(phi_W[:, 2 * D:, :]).at[:, 16, :].set(phi_b)
    g0 = gam_W[:, :D, :]
    g1 = gam_W[:, D:, :]
    gb8 = jnp.broadcast_to(gam_b[:, None, :], (L, 8, D))

    sc_segsum = _make_sc_segsum()

    e_parts = sc_segsum(e2, eidx3, row3, zs)
    e0, e1 = e_parts[0], e_parts[1]
    for l in range(L):
        s_parts = sc_segsum(h, col3, row3, zs)
        h = _tc_layer(h, s_parts[0], s_parts[1], e0, e1,
                      ab[l], bw[l], cx[l], g0[l], g1[l], gb8[l],
                      residual=(l < L - 1))
    return h[:N]


# speedup vs baseline: 3.4996x; 1.0648x over previous
import functools

import jax
import jax.numpy as jnp
from jax import lax
from jax.experimental import pallas as pl
from jax.experimental.pallas import tpu as pltpu
from jax.experimental.pallas import tpu_sc as plsc

N = 10000
E = 320000
D = 128
N_PAD = 10240
E_PAD = 327680
N_TILES = 16
N_CORES = 2
ROWS_PER_TILE = N_PAD // N_TILES
CHUNK = 128
CHUNKS_PER_TILE = E_PAD // (N_TILES * N_CORES * CHUNK)
NBUF = 2
N_PHASES = 2
PHASE_CHUNKS = CHUNKS_PER_TILE // N_PHASES
f32 = jnp.float32


def _make_sc_segsum():
    mesh = plsc.VectorSubcoreMesh(core_axis_name="c", subcore_axis_name="s")

    @functools.partial(
        pl.kernel, mesh=mesh,
        out_type=jax.ShapeDtypeStruct((N_CORES, N_PAD, D), f32),
        scratch_types=[
            pltpu.VMEM((PHASE_CHUNKS, CHUNK), jnp.int32),
            pltpu.VMEM((PHASE_CHUNKS, CHUNK), jnp.int32),
            *[pltpu.VMEM((CHUNK, D), f32) for _ in range(NBUF)],
            pltpu.VMEM_SHARED((N_PAD, D), f32),
            *[pltpu.SemaphoreType.DMA for _ in range(NBUF)],
        ])
    def sc_kernel(tab_hbm, idx3, row3, zs, s_out, idxbuf, rowbuf, *rest):
        bufs = rest[:NBUF]
        sacc = rest[NBUF]
        sems = rest[NBUF + 1:]
        cid = lax.axis_index("c")
        sid = lax.axis_index("s")
        wid = cid * N_TILES + sid
        rbase = sid * ROWS_PER_TILE
        pltpu.sync_copy(zs, sacc.at[pl.ds(rbase, ROWS_PER_TILE)])
        plsc.subcore_barrier()

        for phase in range(N_PHASES):
            pltpu.sync_copy(idx3.at[wid, pl.ds(phase * PHASE_CHUNKS,
                                               PHASE_CHUNKS)], idxbuf)
            pltpu.sync_copy(row3.at[wid, pl.ds(phase * PHASE_CHUNKS,
                                               PHASE_CHUNKS)], rowbuf)

            for b in range(NBUF):
                pltpu.async_copy(tab_hbm.at[idxbuf.at[b]], bufs[b], sems[b])

            def ring_body(i, carry):
                jbase = i * NBUF
                for b in range(NBUF):
                    pltpu.make_async_copy(tab_hbm.at[pl.ds(0, CHUNK)],
                                          bufs[b], sems[b]).wait()
                    pltpu.sync_copy(bufs[b], sacc.at[rowbuf.at[jbase + b]],
                                    add=True)
                    pltpu.async_copy(tab_hbm.at[idxbuf.at[jbase + NBUF + b]],
                                     bufs[b], sems[b])
                return carry

            lax.fori_loop(0, PHASE_CHUNKS // NBUF - 1, ring_body, 0)
            tail = PHASE_CHUNKS - NBUF
            for b in range(NBUF):
                pltpu.make_async_copy(tab_hbm.at[pl.ds(0, CHUNK)],
                                      bufs[b], sems[b]).wait()
                pltpu.sync_copy(bufs[b], sacc.at[rowbuf.at[tail + b]],
                                add=True)
        plsc.subcore_barrier()
        pltpu.sync_copy(sacc.at[pl.ds(rbase, ROWS_PER_TILE)],
                        s_out.at[cid, pl.ds(rbase, ROWS_PER_TILE)])

    return sc_kernel


def _tc_layer(h, s0, s1, e0, e1, ab, bw, cx, g0, g1, gb8, residual: bool):
    BR = 1024
    grid = (N_PAD // BR,)

    def body(h_ref, s0_ref, s1_ref, e0_ref, e1_ref, ab_ref, bw_ref, cx_ref,
             g0_ref, g1_ref, gb_ref, out_ref):
        eb = e0_ref[...] + e1_ref[...]
        cnt = eb[:, 16:17]
        inv = 1.0 / jnp.maximum(cnt, 1.0)
        msk = (cnt > 0.0).astype(f32)
        sb = (s0_ref[...] + s1_ref[...]) * inv
        me = eb * inv
        hv = h_ref[...]
        aggr = (jnp.dot(hv, ab_ref[...], preferred_element_type=f32)
                + jnp.dot(sb, bw_ref[...], preferred_element_type=f32)
                + jnp.dot(me, cx_ref[...], preferred_element_type=f32)) * msk
        out = (jnp.dot(hv, g0_ref[...], preferred_element_type=f32)
               + jnp.dot(aggr, g1_ref[...], preferred_element_type=f32)
               + gb_ref[0:1, :])
        out_ref[...] = out + hv if residual else out

    rowspec = pl.BlockSpec((BR, D), lambda i: (i, 0))
    wspec = pl.BlockSpec((D, D), lambda i: (0, 0))
    return pl.pallas_call(
        body,
        grid=grid,
        in_specs=[rowspec, rowspec, rowspec, rowspec, rowspec,
                  wspec, wspec, wspec,
                  wspec, wspec, pl.BlockSpec((8, D), lambda i: (0, 0))],
        out_specs=rowspec,
        out_shape=jax.ShapeDtypeStruct((N_PAD, D), f32),
    )(h, s0, s1, e0, e1, ab, bw, cx, g0, g1, gb8)


def kernel(state_node, state_edge, edge_index, phi_W, phi_b, gam_W, gam_b):
    L = phi_W.shape[0]
    h = jnp.pad(state_node, ((0, N_PAD - N), (0, 0)))
    pad = E_PAD - E
    row3 = jnp.pad(edge_index[0], (0, pad), constant_values=N).reshape(
        N_TILES * N_CORES, CHUNKS_PER_TILE, CHUNK)
    col3 = jnp.pad(edge_index[1], (0, pad), constant_values=N).reshape(
        N_TILES * N_CORES, CHUNKS_PER_TILE, CHUNK)
    eidx3 = jnp.arange(E_PAD, dtype=jnp.int32).reshape(
        N_TILES * N_CORES, CHUNKS_PER_TILE, CHUNK)
    e2 = jnp.zeros((E_PAD, D), f32)
    e2 = e2.at[:E, :16].set(state_edge).at[:E, 16].set(1.0)
    zs = jnp.zeros((ROWS_PER_TILE, D), f32)

    ab = phi_W[:, :D, :] - phi_W[:, D:2 * D, :]
    bw = phi_W[:, D:2 * D, :]
    cx = jnp.zeros((L, D, D), f32)
    cx = cx.at[:, :16, :].set(phi_W[:, 2 * D:, :]).at[:, 16, :].set(phi_b)
    g0 = gam_W[:, :D, :]
    g1 = gam_W[:, D:, :]
    gb8 = jnp.broadcast_to(gam_b[:, None, :], (L, 8, D))

    sc_segsum = _make_sc_segsum()

    e_parts = sc_segsum(e2, eidx3, row3, zs)
    e0, e1 = e_parts[0], e_parts[1]
    for l in range(L):
        s_parts = sc_segsum(h, col3, row3, zs)
        h = _tc_layer(h, s_parts[0], s_parts[1], e0, e1,
                      ab[l], bw[l], cx[l], g0[l], g1[l], gb8[l],
                      residual=(l < L - 1))
    return h[:N]


# speedup vs baseline: 6.3879x; 1.8253x over previous
import functools

import jax
import jax.numpy as jnp
from jax import lax
from jax.experimental import pallas as pl
from jax.experimental.pallas import tpu as pltpu
from jax.experimental.pallas import tpu_sc as plsc

N = 10000
E = 320000
D = 128
N_PAD = 10240
E_PAD = 327680
N_TILES = 16
N_CORES = 2
ROWS_PER_TILE = N_PAD // N_TILES
CHUNK = 128
CHUNKS_PER_TILE = E_PAD // (N_TILES * N_CORES * CHUNK)
NBUF = 2
N_PHASES = 2
PHASE_CHUNKS = CHUNKS_PER_TILE // N_PHASES
f32 = jnp.float32


def _make_sc_segsum():
    mesh = plsc.VectorSubcoreMesh(core_axis_name="c", subcore_axis_name="s")

    @functools.partial(
        pl.kernel, mesh=mesh,
        out_type=jax.ShapeDtypeStruct((N_CORES, N_PAD, D), f32),
        scratch_types=[
            pltpu.VMEM((PHASE_CHUNKS, CHUNK), jnp.int32),
            pltpu.VMEM((PHASE_CHUNKS, CHUNK), jnp.int32),
            *[pltpu.VMEM((CHUNK, D), f32) for _ in range(NBUF)],
            pltpu.VMEM_SHARED((N_PAD, D), f32),
            *[pltpu.SemaphoreType.DMA for _ in range(NBUF)],
        ])
    def sc_kernel(tab_hbm, idx3, row3, zs, s_out, idxbuf, rowbuf, *rest):
        bufs = rest[:NBUF]
        sacc = rest[NBUF]
        sems = rest[NBUF + 1:]
        cid = lax.axis_index("c")
        sid = lax.axis_index("s")
        wid = cid * N_TILES + sid
        rbase = sid * ROWS_PER_TILE
        pltpu.sync_copy(zs, sacc.at[pl.ds(rbase, ROWS_PER_TILE)])
        plsc.subcore_barrier()

        for phase in range(N_PHASES):
            pltpu.sync_copy(idx3.at[wid, pl.ds(phase * PHASE_CHUNKS,
                                               PHASE_CHUNKS)], idxbuf)
            pltpu.sync_copy(row3.at[wid, pl.ds(phase * PHASE_CHUNKS,
                                               PHASE_CHUNKS)], rowbuf)

            for b in range(NBUF):
                pltpu.async_copy(tab_hbm.at[idxbuf.at[b]], bufs[b], sems[b])

            def ring_body(i, carry):
                jbase = i * NBUF
                for b in range(NBUF):
                    pltpu.make_async_copy(tab_hbm.at[pl.ds(0, CHUNK)],
                                          bufs[b], sems[b]).wait()
                    pltpu.sync_copy(bufs[b], sacc.at[rowbuf.at[jbase + b]],
                                    add=True)
                    pltpu.async_copy(tab_hbm.at[idxbuf.at[jbase + NBUF + b]],
                                     bufs[b], sems[b])
                return carry

            lax.fori_loop(0, PHASE_CHUNKS // NBUF - 1, ring_body, 0)
            tail = PHASE_CHUNKS - NBUF
            for b in range(NBUF):
                pltpu.make_async_copy(tab_hbm.at[pl.ds(0, CHUNK)],
                                      bufs[b], sems[b]).wait()
                pltpu.sync_copy(bufs[b], sacc.at[rowbuf.at[tail + b]],
                                add=True)
        plsc.subcore_barrier()
        pltpu.sync_copy(sacc.at[pl.ds(rbase, ROWS_PER_TILE)],
                        s_out.at[cid, pl.ds(rbase, ROWS_PER_TILE)])

    return sc_kernel


def _tc_layer(h, s0, s1, e0, e1, ab, bw, cx, g0, g1, gb8, residual: bool):
    BR = 1024
    grid = (N_PAD // BR,)

    def body(h_ref, s0_ref, s1_ref, e0_ref, e1_ref, ab_ref, bw_ref, cx_ref,
             g0_ref, g1_ref, gb_ref, out_ref):
        eb = e0_ref[...] + e1_ref[...]
        cnt = eb[:, 16:17]
        inv = 1.0 / jnp.maximum(cnt, 1.0)
        msk = (cnt > 0.0).astype(f32)
        sb = (s0_ref[...] + s1_ref[...]) * inv
        me = eb * inv
        hv = h_ref[...]
        aggr = (jnp.dot(hv, ab_ref[...], preferred_element_type=f32)
                + jnp.dot(sb, bw_ref[...], preferred_element_type=f32)
                + jnp.dot(me, cx_ref[...], preferred_element_type=f32)) * msk
        out = (jnp.dot(hv, g0_ref[...], preferred_element_type=f32)
               + jnp.dot(aggr, g1_ref[...], preferred_element_type=f32)
               + gb_ref[0:1, :])
        out_ref[...] = out + hv if residual else out

    rowspec = pl.BlockSpec((BR, D), lambda i: (i, 0))
    wspec = pl.BlockSpec((D, D), lambda i: (0, 0))
    return pl.pallas_call(
        body,
        grid=grid,
        in_specs=[rowspec, rowspec, rowspec, rowspec, rowspec,
                  wspec, wspec, wspec,
                  wspec, wspec, pl.BlockSpec((8, D), lambda i: (0, 0))],
        out_specs=rowspec,
        out_shape=jax.ShapeDtypeStruct((N_PAD, D), f32),
    )(h, s0, s1, e0, e1, ab, bw, cx, g0, g1, gb8)


def kernel(state_node, state_edge, edge_index, phi_W, phi_b, gam_W, gam_b):
    L = phi_W.shape[0]
    h = jnp.pad(state_node, ((0, N_PAD - N), (0, 0)))
    pad = E_PAD - E
    dummy = N + jnp.arange(pad, dtype=jnp.int32) % (N_PAD - N)
    row3 = jnp.concatenate([edge_index[0], dummy]).reshape(
        N_TILES * N_CORES, CHUNKS_PER_TILE, CHUNK)
    col3 = jnp.concatenate([edge_index[1], dummy]).reshape(
        N_TILES * N_CORES, CHUNKS_PER_TILE, CHUNK)
    eidx3 = jnp.arange(E_PAD, dtype=jnp.int32).reshape(
        N_TILES * N_CORES, CHUNKS_PER_TILE, CHUNK)
    e2 = jnp.zeros((E_PAD, D), f32)
    e2 = e2.at[:E, :16].set(state_edge).at[:E, 16].set(1.0)
    zs = jnp.zeros((ROWS_PER_TILE, D), f32)

    ab = phi_W[:, :D, :] - phi_W[:, D:2 * D, :]
    bw = phi_W[:, D:2 * D, :]
    cx = jnp.zeros((L, D, D), f32)
    cx = cx.at[:, :16, :].set(phi_W[:, 2 * D:, :]).at[:, 16, :].set(phi_b)
    g0 = gam_W[:, :D, :]
    g1 = gam_W[:, D:, :]
    gb8 = jnp.broadcast_to(gam_b[:, None, :], (L, 8, D))

    sc_segsum = _make_sc_segsum()

    e_parts = sc_segsum(e2, eidx3, row3, zs)
    e0, e1 = e_parts[0], e_parts[1]
    for l in range(L):
        s_parts = sc_segsum(h, col3, row3, zs)
        h = _tc_layer(h, s_parts[0], s_parts[1], e0, e1,
                      ab[l], bw[l], cx[l], g0[l], g1[l], gb8[l],
                      residual=(l < L - 1))
    return h[:N]


# speedup vs baseline: 10.5706x; 1.6548x over previous
import functools

import jax
import jax.numpy as jnp
from jax import lax
from jax.experimental import pallas as pl
from jax.experimental.pallas import tpu as pltpu
from jax.experimental.pallas import tpu_sc as plsc

N = 10000
E = 320000
D = 128
N_PAD = 10240
E_PAD = 327680
N_TILES = 16
N_CORES = 2
ROWS_PER_TILE = N_PAD // N_TILES
CHUNK = 128
CHUNKS_PER_TILE = E_PAD // (N_TILES * N_CORES * CHUNK)
NBUF = 2
N_PHASES = 2
PHASE_CHUNKS = CHUNKS_PER_TILE // N_PHASES
f32 = jnp.float32


def _make_sc_segsum():
    mesh = plsc.VectorSubcoreMesh(core_axis_name="c", subcore_axis_name="s")

    @functools.partial(
        pl.kernel, mesh=mesh,
        out_type=jax.ShapeDtypeStruct((N_CORES, N_PAD, D), f32),
        scratch_types=[
            pltpu.VMEM((PHASE_CHUNKS, CHUNK), jnp.int32),
            pltpu.VMEM((PHASE_CHUNKS, CHUNK), jnp.int32),
            *[pltpu.VMEM((CHUNK, D), f32) for _ in range(NBUF)],
            pltpu.VMEM_SHARED((N_PAD, D), f32),
            *[pltpu.SemaphoreType.DMA for _ in range(NBUF)],
        ])
    def sc_kernel(tab_hbm, idx3, row3, zs, s_out, idxbuf, rowbuf, *rest):
        bufs = rest[:NBUF]
        sacc = rest[NBUF]
        sems = rest[NBUF + 1:]
        cid = lax.axis_index("c")
        sid = lax.axis_index("s")
        wid = cid * N_TILES + sid
        rbase = sid * ROWS_PER_TILE
        pltpu.sync_copy(zs, sacc.at[pl.ds(rbase, ROWS_PER_TILE)])
        plsc.subcore_barrier()

        for phase in range(N_PHASES):
            pltpu.sync_copy(idx3.at[wid, pl.ds(phase * PHASE_CHUNKS,
                                               PHASE_CHUNKS)], idxbuf)
            pltpu.sync_copy(row3.at[wid, pl.ds(phase * PHASE_CHUNKS,
                                               PHASE_CHUNKS)], rowbuf)

            for b in range(NBUF):
                pltpu.async_copy(tab_hbm.at[idxbuf.at[b]], bufs[b], sems[b])

            def ring_body(i, carry):
                jbase = i * NBUF
                for b in range(NBUF):
                    pltpu.make_async_copy(tab_hbm.at[pl.ds(0, CHUNK)],
                                          bufs[b], sems[b]).wait()
                    pltpu.sync_copy(bufs[b], sacc.at[rowbuf.at[jbase + b]],
                                    add=True)
                    pltpu.async_copy(tab_hbm.at[idxbuf.at[jbase + NBUF + b]],
                                     bufs[b], sems[b])
                return carry

            lax.fori_loop(0, PHASE_CHUNKS // NBUF - 1, ring_body, 0)
            tail = PHASE_CHUNKS - NBUF
            for b in range(NBUF):
                pltpu.make_async_copy(tab_hbm.at[pl.ds(0, CHUNK)],
                                      bufs[b], sems[b]).wait()
                pltpu.sync_copy(bufs[b], sacc.at[rowbuf.at[tail + b]],
                                add=True)
        plsc.subcore_barrier()
        pltpu.sync_copy(sacc.at[pl.ds(rbase, ROWS_PER_TILE)],
                        s_out.at[cid, pl.ds(rbase, ROWS_PER_TILE)])

    return sc_kernel


def _tc_layer(h, s0, s1, e0, e1, ab, bw, cx, g0, g1, gb8, residual: bool):
    BR = 1024
    grid = (N_PAD // BR,)

    def body(h_ref, s0_ref, s1_ref, e0_ref, e1_ref, ab_ref, bw_ref, cx_ref,
             g0_ref, g1_ref, gb_ref, out_ref):
        eb = e0_ref[...] + e1_ref[...]
        cnt = eb[:, 16:17]
        inv = 1.0 / jnp.maximum(cnt, 1.0)
        msk = (cnt > 0.0).astype(f32)
        sb = (s0_ref[...] + s1_ref[...]) * inv
        me = eb * inv
        hv = h_ref[...]
        aggr = (jnp.dot(hv, ab_ref[...], preferred_element_type=f32)
                + jnp.dot(sb, bw_ref[...], preferred_element_type=f32)
                + jnp.dot(me, cx_ref[...], preferred_element_type=f32)) * msk
        out = (jnp.dot(hv, g0_ref[...], preferred_element_type=f32)
               + jnp.dot(aggr, g1_ref[...], preferred_element_type=f32)
               + gb_ref[0:1, :])
        out_ref[...] = out + hv if residual else out

    rowspec = pl.BlockSpec((BR, D), lambda i: (i, 0))
    wspec = pl.BlockSpec((D, D), lambda i: (0, 0))
    return pl.pallas_call(
        body,
        grid=grid,
        in_specs=[rowspec, rowspec, rowspec, rowspec, rowspec,
                  wspec, wspec, wspec,
                  wspec, wspec, pl.BlockSpec((8, D), lambda i: (0, 0))],
        out_specs=rowspec,
        out_shape=jax.ShapeDtypeStruct((N_PAD, D), f32),
    )(h, s0, s1, e0, e1, ab, bw, cx, g0, g1, gb8)


def kernel(state_node, state_edge, edge_index, phi_W, phi_b, gam_W, gam_b):
    L = phi_W.shape[0]
    h = jnp.pad(state_node, ((0, N_PAD - N), (0, 0)))
    pad = E_PAD - E
    dummy = N + jnp.arange(pad, dtype=jnp.int32) % (N_PAD - N)
    row3 = jnp.concatenate([edge_index[0], dummy]).reshape(
        N_TILES * N_CORES, CHUNKS_PER_TILE, CHUNK)
    col3 = jnp.concatenate([edge_index[1], dummy]).reshape(
        N_TILES * N_CORES, CHUNKS_PER_TILE, CHUNK)
    eidx3 = jnp.arange(E_PAD, dtype=jnp.int32).reshape(
        N_TILES * N_CORES, CHUNKS_PER_TILE, CHUNK)
    e2 = jnp.pad(
        jnp.concatenate(
            [state_edge, jnp.ones((E, 1), f32), jnp.zeros((E, D - 17), f32)],
            axis=1),
        ((0, pad), (0, 0)))
    zs = jnp.zeros((ROWS_PER_TILE, D), f32)

    ab = phi_W[:, :D, :] - phi_W[:, D:2 * D, :]
    bw = phi_W[:, D:2 * D, :]
    cx = jnp.zeros((L, D, D), f32)
    cx = cx.at[:, :16, :].set(phi_W[:, 2 * D:, :]).at[:, 16, :].set(phi_b)
    g0 = gam_W[:, :D, :]
    g1 = gam_W[:, D:, :]
    gb8 = jnp.broadcast_to(gam_b[:, None, :], (L, 8, D))

    sc_segsum = _make_sc_segsum()

    e_parts = sc_segsum(e2, eidx3, row3, zs)
    e0, e1 = e_parts[0], e_parts[1]
    for l in range(L):
        s_parts = sc_segsum(h, col3, row3, zs)
        h = _tc_layer(h, s_parts[0], s_parts[1], e0, e1,
                      ab[l], bw[l], cx[l], g0[l], g1[l], gb8[l],
                      residual=(l < L - 1))
    return h[:N]


# speedup vs baseline: 10.5754x; 1.0005x over previous
import functools

import jax
import jax.numpy as jnp
from jax import lax
from jax.experimental import pallas as pl
from jax.experimental.pallas import tpu as pltpu
from jax.experimental.pallas import tpu_sc as plsc

N = 10000
E = 320000
D = 128
N_PAD = 10240
E_PAD = 327680
N_TILES = 16
N_CORES = 2
ROWS_PER_TILE = N_PAD // N_TILES
CHUNK = 128
CHUNKS_PER_TILE = E_PAD // (N_TILES * N_CORES * CHUNK)
NBUF = 2
N_PHASES = 2
PHASE_CHUNKS = CHUNKS_PER_TILE // N_PHASES
f32 = jnp.float32


def _make_sc_segsum():
    mesh = plsc.VectorSubcoreMesh(core_axis_name="c", subcore_axis_name="s")

    @functools.partial(
        pl.kernel, mesh=mesh,
        out_type=jax.ShapeDtypeStruct((N_CORES, N_PAD, D), f32),
        scratch_types=[
            pltpu.VMEM((PHASE_CHUNKS, CHUNK), jnp.int32),
            pltpu.VMEM((PHASE_CHUNKS, CHUNK), jnp.int32),
            *[pltpu.VMEM((CHUNK, D), f32) for _ in range(NBUF)],
            pltpu.VMEM_SHARED((N_PAD, D), f32),
            *[pltpu.SemaphoreType.DMA for _ in range(NBUF)],
        ])
    def sc_kernel(tab_hbm, idx3, row3, zs, s_out, idxbuf, rowbuf, *rest):
        bufs = rest[:NBUF]
        sacc = rest[NBUF]
        sems = rest[NBUF + 1:]
        cid = lax.axis_index("c")
        sid = lax.axis_index("s")
        wid = cid * N_TILES + sid
        rbase = sid * ROWS_PER_TILE
        pltpu.sync_copy(zs, sacc.at[pl.ds(rbase, ROWS_PER_TILE)])
        plsc.subcore_barrier()

        for phase in range(N_PHASES):
            pltpu.sync_copy(idx3.at[wid, pl.ds(phase * PHASE_CHUNKS,
                                               PHASE_CHUNKS)], idxbuf)
            pltpu.sync_copy(row3.at[wid, pl.ds(phase * PHASE_CHUNKS,
                                               PHASE_CHUNKS)], rowbuf)

            for b in range(NBUF):
                pltpu.async_copy(tab_hbm.at[idxbuf.at[b]], bufs[b], sems[b])

            def ring_body(i, carry):
                jbase = i * NBUF
                for b in range(NBUF):
                    pltpu.make_async_copy(tab_hbm.at[pl.ds(0, CHUNK)],
                                          bufs[b], sems[b]).wait()
                    pltpu.sync_copy(bufs[b], sacc.at[rowbuf.at[jbase + b]],
                                    add=True)
                    pltpu.async_copy(tab_hbm.at[idxbuf.at[jbase + NBUF + b]],
                                     bufs[b], sems[b])
                return carry

            lax.fori_loop(0, PHASE_CHUNKS // NBUF - 1, ring_body, 0)
            tail = PHASE_CHUNKS - NBUF
            for b in range(NBUF):
                pltpu.make_async_copy(tab_hbm.at[pl.ds(0, CHUNK)],
                                      bufs[b], sems[b]).wait()
                pltpu.sync_copy(bufs[b], sacc.at[rowbuf.at[tail + b]],
                                add=True)
        plsc.subcore_barrier()
        pltpu.sync_copy(sacc.at[pl.ds(rbase, ROWS_PER_TILE)],
                        s_out.at[cid, pl.ds(rbase, ROWS_PER_TILE)])

    return sc_kernel


def _make_sc_first():
    mesh = plsc.VectorSubcoreMesh(core_axis_name="c", subcore_axis_name="s")
    WCHUNKS = CHUNKS_PER_TILE * N_CORES
    WPHASES = WCHUNKS // PHASE_CHUNKS

    @functools.partial(
        pl.kernel, mesh=mesh,
        out_type=jax.ShapeDtypeStruct((N_CORES, N_PAD, D), f32),
        scratch_types=[
            pltpu.VMEM((PHASE_CHUNKS, CHUNK), jnp.int32),
            pltpu.VMEM((PHASE_CHUNKS, CHUNK), jnp.int32),
            *[pltpu.VMEM((CHUNK, D), f32) for _ in range(NBUF)],
            pltpu.VMEM_SHARED((N_PAD, D), f32),
            *[pltpu.SemaphoreType.DMA for _ in range(NBUF)],
        ])
    def sc_kernel(h_hbm, e2_hbm, col3, eidx3, row3, zs, s_out,
                  idxbuf, rowbuf, *rest):
        bufs = rest[:NBUF]
        sacc = rest[NBUF]
        sems = rest[NBUF + 1:]
        cid = lax.axis_index("c")
        sid = lax.axis_index("s")
        rbase = sid * ROWS_PER_TILE
        pltpu.sync_copy(zs, sacc.at[pl.ds(rbase, ROWS_PER_TILE)])
        plsc.subcore_barrier()

        def run(tab_hbm, idx3):
            for p in range(WPHASES):
                r0 = 2 * sid + (p // 2)
                c0 = (p % 2) * PHASE_CHUNKS
                pltpu.sync_copy(idx3.at[r0, pl.ds(c0, PHASE_CHUNKS)], idxbuf)
                pltpu.sync_copy(row3.at[r0, pl.ds(c0, PHASE_CHUNKS)], rowbuf)
                for b in range(NBUF):
                    pltpu.async_copy(tab_hbm.at[idxbuf.at[b]], bufs[b],
                                     sems[b])

                def ring_body(i, carry):
                    jbase = i * NBUF
                    for b in range(NBUF):
                        pltpu.make_async_copy(tab_hbm.at[pl.ds(0, CHUNK)],
                                              bufs[b], sems[b]).wait()
                        pltpu.sync_copy(bufs[b],
                                        sacc.at[rowbuf.at[jbase + b]],
                                        add=True)
                        pltpu.async_copy(
                            tab_hbm.at[idxbuf.at[jbase + NBUF + b]],
                            bufs[b], sems[b])
                    return carry

                lax.fori_loop(0, PHASE_CHUNKS // NBUF - 1, ring_body, 0)
                tail = PHASE_CHUNKS - NBUF
                for b in range(NBUF):
                    pltpu.make_async_copy(tab_hbm.at[pl.ds(0, CHUNK)],
                                          bufs[b], sems[b]).wait()
                    pltpu.sync_copy(bufs[b], sacc.at[rowbuf.at[tail + b]],
                                    add=True)

        @pl.when(cid == 0)
        def _():
            run(h_hbm, col3)

        @pl.when(cid == 1)
        def _():
            run(e2_hbm, eidx3)

        plsc.subcore_barrier()
        pltpu.sync_copy(sacc.at[pl.ds(rbase, ROWS_PER_TILE)],
                        s_out.at[cid, pl.ds(rbase, ROWS_PER_TILE)])

    return sc_kernel


def _tc_layer(h, s_arr, s_terms, e_arr, ab, bw, cx, g0, g1, gb8,
              residual: bool):
    BR = 1024
    grid = (N_PAD // BR,)
    e_base = e_arr.shape[0] - 1

    def body(h_ref, s_ref, e_ref, ab_ref, bw_ref, cx_ref,
             g0_ref, g1_ref, gb_ref, out_ref):
        eb = e_ref[0]
        sb = s_ref[0]
        for k in range(1, s_terms):
            sb = sb + s_ref[k]
        cnt = eb[:, 16:17]
        inv = 1.0 / jnp.maximum(cnt, 1.0)
        msk = (cnt > 0.0).astype(f32)
        sb = sb * inv
        me = eb * inv
        hv = h_ref[...]
        aggr = (jnp.dot(hv, ab_ref[...], preferred_element_type=f32)
                + jnp.dot(sb, bw_ref[...], preferred_element_type=f32)
                + jnp.dot(me, cx_ref[...], preferred_element_type=f32)) * msk
        out = (jnp.dot(hv, g0_ref[...], preferred_element_type=f32)
               + jnp.dot(aggr, g1_ref[...], preferred_element_type=f32)
               + gb_ref[0:1, :])
        out_ref[...] = out + hv if residual else out

    rowspec = pl.BlockSpec((BR, D), lambda i: (i, 0))
    wspec = pl.BlockSpec((D, D), lambda i: (0, 0))
    sspec = pl.BlockSpec((s_terms, BR, D), lambda i: (0, i, 0))
    espec = pl.BlockSpec((1, BR, D), lambda i: (e_base, i, 0))
    return pl.pallas_call(
        body,
        grid=grid,
        in_specs=[rowspec, sspec, espec,
                  wspec, wspec, wspec,
                  wspec, wspec, pl.BlockSpec((8, D), lambda i: (0, 0))],
        out_specs=rowspec,
        out_shape=jax.ShapeDtypeStruct((N_PAD, D), f32),
    )(h, s_arr, e_arr, ab, bw, cx, g0, g1, gb8)


def kernel(state_node, state_edge, edge_index, phi_W, phi_b, gam_W, gam_b):
    L = phi_W.shape[0]
    h = jnp.pad(state_node, ((0, N_PAD - N), (0, 0)))
    pad = E_PAD - E
    dummy = N + jnp.arange(pad, dtype=jnp.int32) % (N_PAD - N)
    row3 = jnp.concatenate([edge_index[0], dummy]).reshape(
        N_TILES * N_CORES, CHUNKS_PER_TILE, CHUNK)
    col3 = jnp.concatenate([edge_index[1], dummy]).reshape(
        N_TILES * N_CORES, CHUNKS_PER_TILE, CHUNK)
    eidx3 = jnp.arange(E_PAD, dtype=jnp.int32).reshape(
        N_TILES * N_CORES, CHUNKS_PER_TILE, CHUNK)
    e2 = jnp.pad(
        jnp.concatenate(
            [state_edge, jnp.ones((E, 1), f32), jnp.zeros((E, D - 17), f32)],
            axis=1),
        ((0, pad), (0, 0)))
    zs = jnp.zeros((ROWS_PER_TILE, D), f32)

    ab = phi_W[:, :D, :] - phi_W[:, D:2 * D, :]
    bw = phi_W[:, D:2 * D, :]
    cx = jnp.zeros((L, D, D), f32)
    cx = cx.at[:, :16, :].set(phi_W[:, 2 * D:, :]).at[:, 16, :].set(phi_b)
    g0 = gam_W[:, :D, :]
    g1 = gam_W[:, D:, :]
    gb8 = jnp.broadcast_to(gam_b[:, None, :], (L, 8, D))

    sc_segsum = _make_sc_segsum()
    sc_first = _make_sc_first()

    first = sc_first(h, e2, col3, eidx3, row3, zs)
    h = _tc_layer(h, first, 1, first, ab[0], bw[0], cx[0], g0[0], g1[0],
                  gb8[0], residual=(L > 1))
    for l in range(1, L):
        s_parts = sc_segsum(h, col3, row3, zs)
        h = _tc_layer(h, s_parts, 2, first,
                      ab[l], bw[l], cx[l], g0[l], g1[l], gb8[l],
                      residual=(l < L - 1))
    return h[:N]
